# baseline (speedup 1.0000x reference)
import functools

import jax
import jax.numpy as jnp
from jax.experimental import pallas as pl
from jax.experimental.pallas import tpu as pltpu

KEEP = 0.1
ROW_BLOCK = 512


def _body(k, x_ref, o_ref):
    x = x_ref[...]
    bits = jax.lax.bitcast_convert_type(x, jnp.int32) & jnp.int32(0x7FFFFFFF)
    codes = (bits >> 16).astype(jnp.int16)
    lo = jnp.zeros((x.shape[0], 1), jnp.int32)
    for b in range(14, -1, -1):
        cand = lo | jnp.int32(1 << b)
        cand16 = cand.astype(jnp.int16)
        cnt16 = jnp.sum((codes >= cand16).astype(jnp.int16), axis=1,
                        keepdims=True)
        cnt = cnt16.astype(jnp.int32)
        lo = jnp.where(cnt >= k, cand, lo)
    lo = lo << 16
    for b in range(15, 10, -1):
        cand = lo | jnp.int32(1 << b)
        cnt = jnp.sum((bits >= cand).astype(jnp.int32), axis=1, keepdims=True)
        lo = jnp.where(cnt >= k, cand, lo)
    thr = jax.lax.bitcast_convert_type(lo, jnp.float32)
    ax = jax.lax.bitcast_convert_type(bits, jnp.float32)
    mask = jax.nn.sigmoid(10.0 * (ax - thr))
    o_ref[...] = x * mask


def kernel(x):
    B, T, D = x.shape
    k = max(1, int(D * KEEP))
    R = B * T
    xr = x.reshape(R, D)
    grid = R // ROW_BLOCK
    out = pl.pallas_call(
        functools.partial(_body, k),
        grid=(grid,),
        in_specs=[pl.BlockSpec((ROW_BLOCK, D), lambda i: (i, 0))],
        out_specs=pl.BlockSpec((ROW_BLOCK, D), lambda i: (i, 0)),
        out_shape=jax.ShapeDtypeStruct((R, D), x.dtype),
    )(xr)
    return out.reshape(B, T, D)


# speedup vs baseline: 1.8377x; 1.8377x over previous
import functools

import jax
import jax.numpy as jnp
from jax.experimental import pallas as pl
from jax.experimental.pallas import tpu as pltpu

KEEP = 0.1
ROW_BLOCK = 512


def _body(k, x_ref, o_ref):
    x = x_ref[...]
    bits = jax.lax.bitcast_convert_type(x, jnp.int32) & jnp.int32(0x7FFFFFFF)
    ones_col = jnp.ones((x.shape[1], 8), jnp.float32)
    kf = jnp.float32(k)
    nchain = 4
    rows = x.shape[0] // nchain
    chunks = [bits[i * rows:(i + 1) * rows] for i in range(nchain)]
    los = [jnp.zeros((rows, 1), jnp.int32) for _ in range(nchain)]
    for b in range(30, 12, -1):
        bit = jnp.int32(1 << b)
        cands = [lo | bit for lo in los]
        inds = [(c >= cand).astype(jnp.float32)
                for c, cand in zip(chunks, cands)]
        cnts = [jax.lax.dot_general(ind, ones_col, (((1,), (0,)), ((), ())),
                                    preferred_element_type=jnp.float32)[:, 0:1]
                for ind in inds]
        los = [jnp.where(cnt >= kf, cand, lo)
               for cnt, cand, lo in zip(cnts, cands, los)]
    lo = jnp.concatenate(los, axis=0)
    thr = jax.lax.bitcast_convert_type(lo, jnp.float32)
    ax = jax.lax.bitcast_convert_type(bits, jnp.float32)
    mask = jax.nn.sigmoid(10.0 * (ax - thr))
    o_ref[...] = x * mask


def kernel(x):
    B, T, D = x.shape
    k = max(1, int(D * KEEP))
    R = B * T
    xr = x.reshape(R, D)
    grid = R // ROW_BLOCK
    out = pl.pallas_call(
        functools.partial(_body, k),
        grid=(grid,),
        in_specs=[pl.BlockSpec((ROW_BLOCK, D), lambda i: (i, 0))],
        out_specs=pl.BlockSpec((ROW_BLOCK, D), lambda i: (i, 0)),
        out_shape=jax.ShapeDtypeStruct((R, D), x.dtype),
    )(xr)
    return out.reshape(B, T, D)
